# Initial kernel scaffold; baseline (speedup 1.0000x reference)
import jax
import jax.numpy as jnp
from jax.experimental import pallas as pl


def kernel(x):
    raise NotImplementedError("write your pallas kernel here")


# speedup vs baseline: 1.0866x; 1.0866x over previous
import functools

import jax
import jax.numpy as jnp
import numpy as np
from jax import lax
from jax.experimental import pallas as pl
from jax.experimental.pallas import tpu as pltpu
from jax.experimental.pallas import tpu_sc as plsc

_SPATIAL_START = [45, 48, 87, 90]
_PATCH_SIZES = [3, 5, 7, 9]

_B, _T, _N, _D = 8, 16, 196, 768
_ROWS = _B * _T * _N
_NC, _NS = 2, 16
_NW = _NC * _NS
_RPW = _ROWS // _NW
_CH = 56
_NCH = _RPW // _CH


def _patch_indices(spatial_point, patch_size):
    sp = spatial_point - 15 * (_PATCH_SIZES.index(patch_size))
    gap = (patch_size + 1) // 2
    additional = [sp, sp + gap, sp + gap * 2,
                  sp + 14 * gap, sp + 14 * gap + gap * 2,
                  sp + 14 * gap * 2, sp + 14 * gap * 2 + gap,
                  sp + 14 * gap * 2 + gap * 2]
    center = [14 * i + sp + j + 1 for j in range(patch_size) for i in range(patch_size)]
    return np.asarray(sorted(additional + center), dtype=np.int64)


def _flat_index_table():
    per_tube = []
    for sp in _SPATIAL_START:
        foff, pids = [], []
        for i, ps in enumerate(_PATCH_SIZES):
            idxs = _patch_indices(sp, ps)
            foff.append(np.full(len(idxs), i, dtype=np.int64))
            pids.append(idxs)
        per_tube.append(np.concatenate(foff) * _N + np.concatenate(pids))
    per_batch = np.concatenate(
        [4 * (t // 4) * _N + per_tube[t % 4] for t in range(_T)])
    idx = np.concatenate([b * _T * _N + per_batch for b in range(_B)])
    return np.asarray(idx, dtype=np.int32)


_IDX = _flat_index_table().reshape(_NW, _NCH, _CH)

@functools.cache
def _build_tube_gather():
    mesh = plsc.VectorSubcoreMesh(
        core_axis_name="c", subcore_axis_name="s",
        num_cores=_NC, num_subcores=_NS)

    @functools.partial(
        pl.kernel,
        out_type=jax.ShapeDtypeStruct((_ROWS, _D), jnp.float32),
        mesh=mesh,
        scratch_types=[
            pltpu.VMEM((_NCH, _CH), jnp.int32),
            pltpu.VMEM((2, _CH, _D), jnp.float32),
            pltpu.SemaphoreType.DMA,
            pltpu.SemaphoreType.DMA,
            pltpu.SemaphoreType.DMA,
            pltpu.SemaphoreType.DMA,
        ],
    )
    def _tube_gather(x_hbm, idx_hbm, out_hbm, idx_v, buf_v,
                     gsem0, gsem1, ssem0, ssem1):
        wid = lax.axis_index("s") * _NC + lax.axis_index("c")
        base = wid * _RPW
        gsem = (gsem0, gsem1)
        ssem = (ssem0, ssem1)

        pltpu.sync_copy(idx_hbm.at[wid], idx_v)

        def start_gather(c):
            return pltpu.async_copy(
                x_hbm.at[idx_v.at[c]], buf_v.at[c % 2], gsem[c % 2])

        def start_store(c):
            return pltpu.async_copy(
                buf_v.at[c % 2], out_hbm.at[pl.ds(base + c * _CH, _CH)],
                ssem[c % 2])

        gathers = [None] * _NCH
        stores = [None] * _NCH
        gathers[0] = start_gather(0)
        for c in range(_NCH):
            gathers[c].wait()
            stores[c] = start_store(c)
            if c + 1 < _NCH:
                if c >= 1:
                    stores[c - 1].wait()
                gathers[c + 1] = start_gather(c + 1)
        stores[_NCH - 2].wait()
        stores[_NCH - 1].wait()

    return _tube_gather


def kernel(x):
    out = _build_tube_gather()(x.reshape(_ROWS, _D), jnp.asarray(_IDX))
    return out.reshape(_B, _T, _N, _D)


# speedup vs baseline: 1.3256x; 1.2199x over previous
import functools

import jax
import jax.numpy as jnp
import numpy as np
from jax import lax
from jax.experimental import pallas as pl
from jax.experimental.pallas import tpu as pltpu
from jax.experimental.pallas import tpu_sc as plsc

_SPATIAL_START = [45, 48, 87, 90]
_PATCH_SIZES = [3, 5, 7, 9]

_B, _T, _N, _D = 8, 16, 196, 768
_ROWS = _B * _T * _N
_NC, _NS = 2, 16
_SLAB = 4 * _N
_CHUNKS = [(0, 16), (16, 8), (24, 24), (48, 8), (56, 48), (104, 8),
           (112, 48), (160, 32), (192, 4)]


def _patch_indices(spatial_point, patch_size):
    sp = spatial_point - 15 * (_PATCH_SIZES.index(patch_size))
    gap = (patch_size + 1) // 2
    additional = [sp, sp + gap, sp + gap * 2,
                  sp + 14 * gap, sp + 14 * gap + gap * 2,
                  sp + 14 * gap * 2, sp + 14 * gap * 2 + gap,
                  sp + 14 * gap * 2 + gap * 2]
    center = [14 * i + sp + j + 1 for j in range(patch_size) for i in range(patch_size)]
    return np.asarray(sorted(additional + center), dtype=np.int64)


def _chunk_tables():
    slab_idx = []
    for sp in _SPATIAL_START:
        fr = np.concatenate([
            np.full(len(_patch_indices(sp, ps)), i, dtype=np.int64)
            for i, ps in enumerate(_PATCH_SIZES)])
        pid = np.concatenate([_patch_indices(sp, ps) for ps in _PATCH_SIZES])
        slab_idx.append(fr * _N + pid)
    slab_idx = np.stack(slab_idx).astype(np.int32)
    return [slab_idx[:, k0:k0 + ln].copy() for (k0, ln) in _CHUNKS]


_CHUNK_IDX = _chunk_tables()


@functools.cache
def _build_tube_gather():
    mesh = plsc.VectorSubcoreMesh(
        core_axis_name="c", subcore_axis_name="s",
        num_cores=_NC, num_subcores=_NS)

    idx_scratch = [pltpu.VMEM(a.shape, jnp.int32) for a in _CHUNK_IDX]
    buf_keys = ["A0", "A1", "C"]
    buf_scratch = [
        pltpu.VMEM((48, _D), jnp.float32),
        pltpu.VMEM((48, _D), jnp.float32),
        pltpu.VMEM((4, _D), jnp.float32),
    ]

    @functools.partial(
        pl.kernel,
        out_type=jax.ShapeDtypeStruct((_B, _T, _N, _D), jnp.float32),
        mesh=mesh,
        scratch_types=(idx_scratch + buf_scratch
                       + [pltpu.SemaphoreType.DMA] * (2 * len(buf_keys))),
    )
    def _tube_gather(x_hbm, *rest):
        nc = len(_CHUNK_IDX)
        idx_in = rest[:nc]
        out_hbm = rest[nc]
        sc = list(rest[nc + 1:])
        idx_v = sc[:nc]
        bufs = dict(zip(buf_keys, sc[nc:nc + len(buf_keys)]))
        sems = sc[nc + len(buf_keys):]
        gsem = dict(zip(buf_keys, sems[:len(buf_keys)]))
        ssem = dict(zip(buf_keys, sems[len(buf_keys):]))

        wid = lax.axis_index("s") * _NC + lax.axis_index("c")
        b = wid // 4
        blk = wid % 4
        t0 = 4 * blk
        slab0 = pl.multiple_of((16 * b + t0) * _N, _SLAB)
        slab = x_hbm.at[pl.ds(slab0, _SLAB)]

        for src, dst in zip(idx_in, idx_v):
            pltpu.sync_copy(src, dst)

        units = []
        for j in range(4):
            for u, (k0, ln) in enumerate(_CHUNKS):
                key = "C" if ln == 4 else f"A{u % 2}"

                def g_u(key=key, j=j, u=u, ln=ln):
                    dst = bufs[key] if ln == 4 else bufs[key].at[pl.ds(0, ln)]
                    return pltpu.async_copy(
                        slab.at[idx_v[u].at[j]], dst, gsem[key])

                def s_u(key=key, j=j, k0=k0, ln=ln):
                    src = bufs[key] if ln == 4 else bufs[key].at[pl.ds(0, ln)]
                    return pltpu.async_copy(
                        src, out_hbm.at[b, t0 + j, pl.ds(k0, ln), :],
                        ssem[key])

                units.append((key, g_u, s_u))

        last_store = {}
        gathers = [None] * len(units)

        def issue(i):
            key = units[i][0]
            h = last_store.pop(key, None)
            if h is not None:
                h.wait()
            gathers[i] = units[i][1]()

        issue(0)
        for i, (key, _, store) in enumerate(units):
            if i + 1 < len(units):
                issue(i + 1)
            gathers[i].wait()
            last_store[key] = store()
        for h in last_store.values():
            h.wait()

    return _tube_gather


def kernel(x):
    args = [jnp.asarray(a) for a in _CHUNK_IDX]
    return _build_tube_gather()(x.reshape(_ROWS, _D), *args)


# speedup vs baseline: 1.3860x; 1.0455x over previous
import functools

import jax
import jax.numpy as jnp
import numpy as np
from jax import lax
from jax.experimental import pallas as pl
from jax.experimental.pallas import tpu as pltpu
from jax.experimental.pallas import tpu_sc as plsc

_SPATIAL_START = [45, 48, 87, 90]
_PATCH_SIZES = [3, 5, 7, 9]

_B, _T, _N, _D = 8, 16, 196, 768
_ROWS = _B * _T * _N
_NC, _NS = 2, 16
_SLAB = 4 * _N
_CHUNKS = [(0, 48), (48, 48), (96, 48), (144, 52)]


def _patch_indices(spatial_point, patch_size):
    sp = spatial_point - 15 * (_PATCH_SIZES.index(patch_size))
    gap = (patch_size + 1) // 2
    additional = [sp, sp + gap, sp + gap * 2,
                  sp + 14 * gap, sp + 14 * gap + gap * 2,
                  sp + 14 * gap * 2, sp + 14 * gap * 2 + gap,
                  sp + 14 * gap * 2 + gap * 2]
    center = [14 * i + sp + j + 1 for j in range(patch_size) for i in range(patch_size)]
    return np.asarray(sorted(additional + center), dtype=np.int64)


def _chunk_tables():
    slab_idx = []
    for sp in _SPATIAL_START:
        fr = np.concatenate([
            np.full(len(_patch_indices(sp, ps)), i, dtype=np.int64)
            for i, ps in enumerate(_PATCH_SIZES)])
        pid = np.concatenate([_patch_indices(sp, ps) for ps in _PATCH_SIZES])
        slab_idx.append(fr * _N + pid)
    slab_idx = np.stack(slab_idx).astype(np.int32)
    return [slab_idx[:, k0:k0 + ln].copy() for (k0, ln) in _CHUNKS]


_CHUNK_IDX = _chunk_tables()


@functools.cache
def _build_tube_gather():
    mesh = plsc.VectorSubcoreMesh(
        core_axis_name="c", subcore_axis_name="s",
        num_cores=_NC, num_subcores=_NS)

    idx_scratch = [pltpu.VMEM(a.shape, jnp.int32) for a in _CHUNK_IDX]
    buf_keys = ["A0", "A1", "A2"]
    buf_scratch = [
        pltpu.VMEM((52, _D), jnp.float32),
        pltpu.VMEM((52, _D), jnp.float32),
        pltpu.VMEM((52, _D), jnp.float32),
    ]

    @functools.partial(
        pl.kernel,
        out_type=jax.ShapeDtypeStruct((_B, _T, _N, _D), jnp.float32),
        mesh=mesh,
        scratch_types=(idx_scratch + buf_scratch
                       + [pltpu.SemaphoreType.DMA] * (2 * len(buf_keys))),
    )
    def _tube_gather(x_hbm, *rest):
        nc = len(_CHUNK_IDX)
        idx_in = rest[:nc]
        out_hbm = rest[nc]
        sc = list(rest[nc + 1:])
        idx_v = sc[:nc]
        bufs = dict(zip(buf_keys, sc[nc:nc + len(buf_keys)]))
        sems = sc[nc + len(buf_keys):]
        gsem = dict(zip(buf_keys, sems[:len(buf_keys)]))
        ssem = dict(zip(buf_keys, sems[len(buf_keys):]))

        wid = lax.axis_index("s") * _NC + lax.axis_index("c")
        b = wid // 4
        blk = wid % 4
        t0 = 4 * blk
        slab0 = pl.multiple_of((16 * b + t0) * _N, _SLAB)
        slab = x_hbm.at[pl.ds(slab0, _SLAB)]

        for src, dst in zip(idx_in, idx_v):
            pltpu.sync_copy(src, dst)

        units = []
        for j in range(4):
            for u, (k0, ln) in enumerate(_CHUNKS):
                key = f"A{len(units) % 3}"

                def g_u(key=key, j=j, u=u, ln=ln):
                    dst = bufs[key] if ln == 52 else bufs[key].at[pl.ds(0, ln)]
                    return pltpu.async_copy(
                        slab.at[idx_v[u].at[j]], dst, gsem[key])

                def s_u(key=key, j=j, k0=k0, ln=ln):
                    src = bufs[key] if ln == 52 else bufs[key].at[pl.ds(0, ln)]
                    return pltpu.async_copy(
                        src, out_hbm.at[b, t0 + j, pl.ds(k0, ln), :],
                        ssem[key])

                units.append((key, g_u, s_u))

        last_store = {}
        gathers = [None] * len(units)

        def issue(i):
            key = units[i][0]
            h = last_store.pop(key, None)
            if h is not None:
                h.wait()
            gathers[i] = units[i][1]()

        la = 2
        for i in range(la):
            issue(i)
        for i, (key, _, store) in enumerate(units):
            if i + la < len(units):
                issue(i + la)
            gathers[i].wait()
            last_store[key] = store()
        for h in last_store.values():
            h.wait()

    return _tube_gather


def kernel(x):
    args = [jnp.asarray(a) for a in _CHUNK_IDX]
    return _build_tube_gather()(x.reshape(_ROWS, _D), *args)
